# Initial kernel scaffold; baseline (speedup 1.0000x reference)
import jax
import jax.numpy as jnp
from jax.experimental import pallas as pl


def kernel(feat_src, orderInfo, edge_weight):
    raise NotImplementedError("write your pallas kernel here")


# speedup vs baseline: 1.3502x; 1.3502x over previous
import functools

import jax
import jax.numpy as jnp
from jax import lax
from jax.experimental import pallas as pl
from jax.experimental.pallas import tpu as pltpu
from jax.experimental.pallas import tpu_sc as plsc

E = 160000
K = 2
D = 128
EK = E * K

NC = 2
NS = 16
NW = NC * NS
PER_W = EK // NW
CR = 400
NCHUNK = PER_W // CR


def _splat_lane(vec, lane):
    return lax.gather(
        vec,
        jnp.full((16, 1), lane, jnp.int32),
        lax.GatherDimensionNumbers(
            offset_dims=(), collapsed_slice_dims=(0,), start_index_map=(0,)),
        slice_sizes=(1,),
        mode=lax.GatherScatterMode.PROMISE_IN_BOUNDS)


@functools.partial(
    pl.kernel,
    out_type=jax.ShapeDtypeStruct((EK, D), jnp.float32),
    mesh=plsc.VectorSubcoreMesh(core_axis_name="c", subcore_axis_name="s"),
    scratch_types=[
        pltpu.VMEM((CR, D), jnp.float32),
        pltpu.VMEM((CR,), jnp.int32),
        pltpu.VMEM((CR,), jnp.float32),
    ],
)
def _sc_message(feat_hbm, oi_hbm, w_hbm, out_hbm, buf, oi_v, w_v):
    wid = lax.axis_index("s") * NC + lax.axis_index("c")
    base = wid * PER_W

    @pl.loop(0, NCHUNK)
    def _chunk(c):
        cb = base + c * CR
        pltpu.sync_copy(feat_hbm.at[pl.ds(cb, CR)], buf)
        pltpu.sync_copy(oi_hbm.at[pl.ds(cb, CR)], oi_v)
        pltpu.sync_copy(w_hbm.at[pl.ds(cb, CR)], w_v)

        @pl.loop(0, CR // 16)
        def _group(g):
            gb = g * 16
            ovec = oi_v[pl.ds(gb, 16)]
            wvec = w_v[pl.ds(gb, 16)]
            avec = ovec.astype(jnp.float32) * wvec
            for el in range(8):
                l0 = 2 * el
                l1 = l0 + 1
                i0 = gb + l0
                i1 = gb + l1
                w0 = _splat_lane(wvec, l0)
                w1 = _splat_lane(wvec, l1)
                a0 = _splat_lane(avec, l0)
                a1 = _splat_lane(avec, l1)
                for j in range(D // 16):
                    sl = pl.ds(j * 16, 16)
                    f0 = buf[i0, sl]
                    f1 = buf[i1, sl]
                    d = f1 - f0
                    buf[i0, sl] = f0 * w0 + d * a0
                    buf[i1, sl] = f0 * w1 + d * a1

        pltpu.sync_copy(buf, out_hbm.at[pl.ds(cb, CR)])


def kernel(feat_src, orderInfo, edge_weight):
    feat = feat_src.reshape(EK, D)
    oi = orderInfo.reshape(EK).astype(jnp.int32)
    w = edge_weight.reshape(EK)
    return _sc_message(feat, oi, w)


# speedup vs baseline: 1.8122x; 1.3422x over previous
import functools

import jax
import jax.numpy as jnp
from jax import lax
from jax.experimental import pallas as pl
from jax.experimental.pallas import tpu as pltpu
from jax.experimental.pallas import tpu_sc as plsc

E = 160000
K = 2
D = 128
EK = E * K

NC = 2
NS = 16
NW = NC * NS
PER_W = EK // NW
CR = 400
NCHUNK = PER_W // CR


def _splat_lane(vec, lane):
    return lax.gather(
        vec,
        jnp.full((16, 1), lane, jnp.int32),
        lax.GatherDimensionNumbers(
            offset_dims=(), collapsed_slice_dims=(0,), start_index_map=(0,)),
        slice_sizes=(1,),
        mode=lax.GatherScatterMode.PROMISE_IN_BOUNDS)


def _compute_chunk(buf, oi_v, w_v):
    @pl.loop(0, CR // 16)
    def _group(g):
        gb = g * 16
        ovec = oi_v[pl.ds(gb, 16)]
        wvec = w_v[pl.ds(gb, 16)]
        avec = ovec.astype(jnp.float32) * wvec
        for el in range(8):
            l0 = 2 * el
            l1 = l0 + 1
            i0 = gb + l0
            i1 = gb + l1
            w0 = _splat_lane(wvec, l0)
            w1 = _splat_lane(wvec, l1)
            a0 = _splat_lane(avec, l0)
            a1 = _splat_lane(avec, l1)
            for j in range(D // 16):
                sl = pl.ds(j * 16, 16)
                f0 = buf[i0, sl]
                f1 = buf[i1, sl]
                d = f1 - f0
                buf[i0, sl] = f0 * w0 + d * a0
                buf[i1, sl] = f0 * w1 + d * a1


PAIRS = NCHUNK // 2


@functools.partial(
    pl.kernel,
    out_type=jax.ShapeDtypeStruct((EK, D), jnp.float32),
    mesh=plsc.VectorSubcoreMesh(core_axis_name="c", subcore_axis_name="s"),
    scratch_types=[
        pltpu.VMEM((CR, D), jnp.float32),
        pltpu.VMEM((CR,), jnp.int32),
        pltpu.VMEM((CR,), jnp.float32),
        pltpu.VMEM((CR, D), jnp.float32),
        pltpu.VMEM((CR,), jnp.int32),
        pltpu.VMEM((CR,), jnp.float32),
        pltpu.SemaphoreType.DMA,
        pltpu.SemaphoreType.DMA,
        pltpu.SemaphoreType.DMA,
        pltpu.SemaphoreType.DMA,
    ],
)
def _sc_message(feat_hbm, oi_hbm, w_hbm, out_hbm,
                buf_a, oi_a, w_a, buf_b, oi_b, w_b,
                ls_a, ls_b, ss_a, ss_b):
    wid = lax.axis_index("s") * NC + lax.axis_index("c")
    base = wid * PER_W

    def start_load(cb, buf, oi_v, w_v, sem):
        pltpu.async_copy(feat_hbm.at[pl.ds(cb, CR)], buf, sem)
        pltpu.async_copy(oi_hbm.at[pl.ds(cb, CR)], oi_v, sem)
        pltpu.async_copy(w_hbm.at[pl.ds(cb, CR)], w_v, sem)

    def wait_load(cb, buf, oi_v, w_v, sem):
        pltpu.make_async_copy(feat_hbm.at[pl.ds(cb, CR)], buf, sem).wait()
        pltpu.make_async_copy(oi_hbm.at[pl.ds(cb, CR)], oi_v, sem).wait()
        pltpu.make_async_copy(w_hbm.at[pl.ds(cb, CR)], w_v, sem).wait()

    start_load(base, buf_a, oi_a, w_a, ls_a)
    start_load(base + CR, buf_b, oi_b, w_b, ls_b)

    @pl.loop(0, PAIRS)
    def _pair(t):
        cb = base + (2 * t) * CR

        wait_load(cb, buf_a, oi_a, w_a, ls_a)
        _compute_chunk(buf_a, oi_a, w_a)
        st_a = pltpu.async_copy(buf_a, out_hbm.at[pl.ds(cb, CR)], ss_a)

        cb2 = cb + CR
        wait_load(cb2, buf_b, oi_b, w_b, ls_b)
        _compute_chunk(buf_b, oi_b, w_b)
        st_b = pltpu.async_copy(buf_b, out_hbm.at[pl.ds(cb2, CR)], ss_b)

        st_a.wait()
        start_load(cb + 2 * CR, buf_a, oi_a, w_a, ls_a)

        st_b.wait()

        @pl.when(t < PAIRS - 1)
        def _():
            start_load(cb + 3 * CR, buf_b, oi_b, w_b, ls_b)

    cb = base + (NCHUNK - 1) * CR
    wait_load(cb, buf_a, oi_a, w_a, ls_a)
    _compute_chunk(buf_a, oi_a, w_a)
    pltpu.async_copy(buf_a, out_hbm.at[pl.ds(cb, CR)], ss_a).wait()


def kernel(feat_src, orderInfo, edge_weight):
    feat = feat_src.reshape(EK, D)
    oi = orderInfo.reshape(EK).astype(jnp.int32)
    w = edge_weight.reshape(EK)
    return _sc_message(feat, oi, w)
